# Initial kernel scaffold; baseline (speedup 1.0000x reference)
import jax
import jax.numpy as jnp
from jax.experimental import pallas as pl


def kernel(x, prior, clip_feature, Wq, bq, Wk, bk, Wg1, bg1, Wg2, bg2, Wfuse, bfuse, Wl1, bl1, Wl2, bl2, Wgate, W1, b1, W2, b2):
    raise NotImplementedError("write your pallas kernel here")


# speedup vs baseline: 3.0271x; 3.0271x over previous
import jax
import jax.numpy as jnp
from jax import lax
from jax.experimental import pallas as pl

C = 96
E = 8
HID = 192
CLIP = 512
H = 224
W = 224
N = H * W
TW = 512
G = N // TW
AO = TW - 256
ATTW = TW + 512


def _ifm_body(xm, x0, xp, pm, p0, pp,
              Wq9, bq, Wk9, bk, Wg19, bg1, Wg29, bg2,
              Wfm, bfuse, clip_col, Wl1T, bl1, Wl2T, bl2, WgT,
              tok_ref, logit_ref):
    t = pl.program_id(0)
    base = t * TW
    win_x = jnp.concatenate([xm[...], x0[...], xp[...]], axis=1)
    win_p = jnp.concatenate([pm[...], p0[...], pp[...]], axis=1)

    def conv9(get_src, W9, b, o0, width, out_c):
        jj = lax.broadcasted_iota(jnp.int32, (1, width), 1)
        p = base + (o0 - TW) + jj
        colp = p % W
        acc = jnp.zeros((out_c, width), jnp.float32)
        for k in range(9):
            dy, dx = k // 3 - 1, k % 3 - 1
            s = dy * W + dx
            q = p + s
            colq = colp + dx
            valid = (q >= 0) & (q < N) & (colq >= 0) & (colq < W)
            src = get_src(o0 + s, width) * valid.astype(jnp.float32)
            acc = acc + jnp.dot(W9[k], src, preferred_element_type=jnp.float32)
        return acc + b[...]

    x_src = lambda off, w: win_x[:, off:off + w]
    p_src = lambda off, w: win_p[:, off:off + w]

    xq = conv9(x_src, Wq9, bq, AO, ATTW, C)
    pk = conv9(p_src, Wk9, bk, AO, ATTW, C)
    att = jax.nn.sigmoid(xq * pk)
    ax = win_x[:, AO:AO + ATTW] * att
    ap = win_p[:, AO:AO + ATTW] * att

    xcat_src = lambda off, w: jnp.concatenate(
        [win_x[:, off:off + w], ax[:, off - AO:off - AO + w]], axis=0)
    pcat_src = lambda off, w: jnp.concatenate(
        [win_p[:, off:off + w], ap[:, off - AO:off - AO + w]], axis=0)

    xg = conv9(xcat_src, Wg19, bg1, TW, TW, 2)
    pg = conv9(pcat_src, Wg29, bg2, TW, TW, 2)

    xc = win_x[:, TW:2 * TW]
    axc = ax[:, TW - AO:2 * TW - AO]
    pc = win_p[:, TW:2 * TW]
    apc = ap[:, TW - AO:2 * TW - AO]
    x_out = xc * xg[0:1] + axc * xg[1:2]
    p_out = pc * pg[0:1] + apc * pg[1:2]

    fused = jnp.dot(Wfm[...], jnp.concatenate([x_out, p_out], axis=0),
                    preferred_element_type=jnp.float32) + bfuse[...]
    m1 = jnp.dot(Wl1T[...], clip_col[...],
                 preferred_element_type=jnp.float32) + bl1[...]
    m2 = jnp.dot(Wl2T[...], clip_col[...],
                 preferred_element_type=jnp.float32) + bl2[...]
    out2 = fused * m1 + m2 + fused
    tok_ref[...] = out2
    logit_ref[...] = jnp.dot(WgT[...], out2, preferred_element_type=jnp.float32)


def _moe_body(tok, logit, W1T, b1, W2T, b2, y_ref):
    lg = logit[...]
    eidx = lax.broadcasted_iota(jnp.int32, (E, TW), 0)
    m1 = jnp.max(lg, axis=0, keepdims=True)
    i1 = jnp.min(jnp.where(lg == m1, eidx, E), axis=0, keepdims=True)
    l2 = jnp.where(eidx == i1, -jnp.inf, lg)
    m2 = jnp.max(l2, axis=0, keepdims=True)
    i2 = jnp.min(jnp.where(l2 == m2, eidx, E), axis=0, keepdims=True)
    texp = jnp.exp(m2 - m1)
    w1 = 1.0 / (1.0 + texp)
    w2 = texp / (1.0 + texp)
    gates = jnp.where(eidx == i1, w1, 0.0) + jnp.where(eidx == i2, w2, 0.0)

    tt = tok[...]
    acc = jnp.zeros((C, TW), jnp.float32)
    for e in range(E):
        h = jnp.dot(W1T[e], tt, preferred_element_type=jnp.float32) + b1[e]
        h = jax.nn.gelu(h)
        o = jnp.dot(W2T[e], h, preferred_element_type=jnp.float32) + b2[e]
        acc = acc + gates[e:e + 1, :] * o
    y_ref[...] = acc


def _full(arr):
    nd = arr.ndim
    return pl.BlockSpec(arr.shape, lambda t: (0,) * nd)


def kernel(x, prior, clip_feature, Wq, bq, Wk, bk, Wg1, bg1, Wg2, bg2,
           Wfuse, bfuse, Wl1, bl1, Wl2, bl2, Wgate, W1, b1, W2, b2):
    xf = x.reshape(C, N)
    pf = prior.reshape(C, N)
    Wq9 = jnp.transpose(Wq, (2, 3, 0, 1)).reshape(9, C, C)
    Wk9 = jnp.transpose(Wk, (2, 3, 0, 1)).reshape(9, C, C)
    Wg19 = jnp.transpose(Wg1, (2, 3, 0, 1)).reshape(9, 2, 2 * C)
    Wg29 = jnp.transpose(Wg2, (2, 3, 0, 1)).reshape(9, 2, 2 * C)
    Wfm = Wfuse.reshape(C, 2 * C)
    clip_col = clip_feature.reshape(CLIP, 1)
    Wl1T = jnp.transpose(Wl1)
    Wl2T = jnp.transpose(Wl2)
    WgT = jnp.transpose(Wgate)
    W1T = jnp.transpose(W1, (0, 2, 1))
    W2T = jnp.transpose(W2, (0, 2, 1))
    b1c = b1.reshape(E, HID, 1)
    b2c = b2.reshape(E, C, 1)
    bqc = bq.reshape(C, 1)
    bkc = bk.reshape(C, 1)
    bg1c = bg1.reshape(2, 1)
    bg2c = bg2.reshape(2, 1)
    bfc = bfuse.reshape(C, 1)
    bl1c = bl1.reshape(C, 1)
    bl2c = bl2.reshape(C, 1)

    blk = lambda im: pl.BlockSpec((C, TW), im)
    tok, logit = pl.pallas_call(
        _ifm_body,
        grid=(G,),
        in_specs=[
            blk(lambda t: (0, jnp.maximum(t - 1, 0))),
            blk(lambda t: (0, t)),
            blk(lambda t: (0, jnp.minimum(t + 1, G - 1))),
            blk(lambda t: (0, jnp.maximum(t - 1, 0))),
            blk(lambda t: (0, t)),
            blk(lambda t: (0, jnp.minimum(t + 1, G - 1))),
            _full(Wq9), _full(bqc), _full(Wk9), _full(bkc),
            _full(Wg19), _full(bg1c), _full(Wg29), _full(bg2c),
            _full(Wfm), _full(bfc), _full(clip_col),
            _full(Wl1T), _full(bl1c), _full(Wl2T), _full(bl2c), _full(WgT),
        ],
        out_specs=[
            pl.BlockSpec((C, TW), lambda t: (0, t)),
            pl.BlockSpec((E, TW), lambda t: (0, t)),
        ],
        out_shape=[
            jax.ShapeDtypeStruct((C, N), jnp.float32),
            jax.ShapeDtypeStruct((E, N), jnp.float32),
        ],
    )(xf, xf, xf, pf, pf, pf,
      Wq9, bqc, Wk9, bkc, Wg19, bg1c, Wg29, bg2c,
      Wfm, bfc, clip_col, Wl1T, bl1c, Wl2T, bl2c, WgT)

    y = pl.pallas_call(
        _moe_body,
        grid=(G,),
        in_specs=[
            pl.BlockSpec((C, TW), lambda t: (0, t)),
            pl.BlockSpec((E, TW), lambda t: (0, t)),
            _full(W1T), _full(b1c), _full(W2T), _full(b2c),
        ],
        out_specs=pl.BlockSpec((C, TW), lambda t: (0, t)),
        out_shape=jax.ShapeDtypeStruct((C, N), jnp.float32),
    )(tok, logit, W1T, b1c, W2T, b2c)

    return y.reshape(1, C, H, W)


# speedup vs baseline: 3.0348x; 1.0025x over previous
import jax
import jax.numpy as jnp
from jax import lax
from jax.experimental import pallas as pl

C = 96
E = 8
HID = 192
CLIP = 512
H = 224
W = 224
N = H * W
TW = 512
G = N // TW
AO = TW - 256
ATTW = TW + 512


def _ifm_body(xm, x0, xp, pm, p0, pp,
              Wq9, bq, Wk9, bk, Wg19, bg1, Wg29, bg2,
              Wfm, bfuse, clip_col, Wl1T, bl1, Wl2T, bl2, WgT,
              tok_ref, logit_ref):
    t = pl.program_id(0)
    base = t * TW
    win_x = jnp.concatenate([xm[...], x0[...], xp[...]], axis=1)
    win_p = jnp.concatenate([pm[...], p0[...], pp[...]], axis=1)

    def conv9(get_src, W9, b, o0, width, out_c):
        jj = lax.broadcasted_iota(jnp.int32, (1, width), 1)
        p = base + (o0 - TW) + jj
        colp = p % W
        acc = jnp.zeros((out_c, width), jnp.float32)
        for k in range(9):
            dy, dx = k // 3 - 1, k % 3 - 1
            s = dy * W + dx
            q = p + s
            colq = colp + dx
            valid = (q >= 0) & (q < N) & (colq >= 0) & (colq < W)
            src = get_src(o0 + s, width) * valid.astype(jnp.float32)
            acc = acc + jnp.dot(W9[k], src, preferred_element_type=jnp.float32)
        return acc + b[...]

    x_src = lambda off, w: win_x[:, off:off + w]
    p_src = lambda off, w: win_p[:, off:off + w]

    xq = conv9(x_src, Wq9, bq, AO, ATTW, C)
    pk = conv9(p_src, Wk9, bk, AO, ATTW, C)
    att = jax.nn.sigmoid(xq * pk)
    ax = win_x[:, AO:AO + ATTW] * att
    ap = win_p[:, AO:AO + ATTW] * att

    xcat_src = lambda off, w: jnp.concatenate(
        [win_x[:, off:off + w], ax[:, off - AO:off - AO + w]], axis=0)
    pcat_src = lambda off, w: jnp.concatenate(
        [win_p[:, off:off + w], ap[:, off - AO:off - AO + w]], axis=0)

    xg = conv9(xcat_src, Wg19, bg1, TW, TW, 2)
    pg = conv9(pcat_src, Wg29, bg2, TW, TW, 2)

    xc = win_x[:, TW:2 * TW]
    axc = ax[:, TW - AO:2 * TW - AO]
    pc = win_p[:, TW:2 * TW]
    apc = ap[:, TW - AO:2 * TW - AO]
    x_out = xc * xg[0:1] + axc * xg[1:2]
    p_out = pc * pg[0:1] + apc * pg[1:2]

    fused = jnp.dot(Wfm[...], jnp.concatenate([x_out, p_out], axis=0),
                    preferred_element_type=jnp.float32) + bfuse[...]
    m1 = jnp.dot(Wl1T[...], clip_col[...],
                 preferred_element_type=jnp.float32) + bl1[...]
    m2 = jnp.dot(Wl2T[...], clip_col[...],
                 preferred_element_type=jnp.float32) + bl2[...]
    out2 = fused * m1 + m2 + fused
    tok_ref[...] = out2.astype(tok_ref.dtype)
    logit_ref[...] = jnp.dot(WgT[...], out2, preferred_element_type=jnp.float32)


def _moe_body(tok, logit, W1T, b1, W2T, b2, y_ref):
    lg = logit[...]
    eidx = lax.broadcasted_iota(jnp.int32, (E, TW), 0)
    m1 = jnp.max(lg, axis=0, keepdims=True)
    i1 = jnp.min(jnp.where(lg == m1, eidx, E), axis=0, keepdims=True)
    l2 = jnp.where(eidx == i1, -jnp.inf, lg)
    m2 = jnp.max(l2, axis=0, keepdims=True)
    i2 = jnp.min(jnp.where(l2 == m2, eidx, E), axis=0, keepdims=True)
    texp = jnp.exp(m2 - m1)
    w1 = 1.0 / (1.0 + texp)
    w2 = texp / (1.0 + texp)
    gates = jnp.where(eidx == i1, w1, 0.0) + jnp.where(eidx == i2, w2, 0.0)

    tt = tok[...]
    acc = jnp.zeros((C, TW), jnp.float32)
    for e in range(E):
        h = jnp.dot(W1T[e], tt, preferred_element_type=jnp.float32) + b1[e]
        h = jax.nn.gelu(h).astype(jnp.bfloat16)
        o = jnp.dot(W2T[e], h, preferred_element_type=jnp.float32) + b2[e]
        acc = acc + gates[e:e + 1, :] * o
    y_ref[...] = acc


def _full(arr):
    nd = arr.ndim
    return pl.BlockSpec(arr.shape, lambda t: (0,) * nd)


def kernel(x, prior, clip_feature, Wq, bq, Wk, bk, Wg1, bg1, Wg2, bg2,
           Wfuse, bfuse, Wl1, bl1, Wl2, bl2, Wgate, W1, b1, W2, b2):
    xf = x.reshape(C, N)
    pf = prior.reshape(C, N)
    Wq9 = jnp.transpose(Wq, (2, 3, 0, 1)).reshape(9, C, C)
    Wk9 = jnp.transpose(Wk, (2, 3, 0, 1)).reshape(9, C, C)
    Wg19 = jnp.transpose(Wg1, (2, 3, 0, 1)).reshape(9, 2, 2 * C)
    Wg29 = jnp.transpose(Wg2, (2, 3, 0, 1)).reshape(9, 2, 2 * C)
    Wfm = Wfuse.reshape(C, 2 * C)
    clip_col = clip_feature.reshape(CLIP, 1)
    Wl1T = jnp.transpose(Wl1)
    Wl2T = jnp.transpose(Wl2)
    WgT = jnp.transpose(Wgate)
    W1T = jnp.transpose(W1, (0, 2, 1)).astype(jnp.bfloat16)
    W2T = jnp.transpose(W2, (0, 2, 1)).astype(jnp.bfloat16)
    b1c = b1.reshape(E, HID, 1)
    b2c = b2.reshape(E, C, 1)
    bqc = bq.reshape(C, 1)
    bkc = bk.reshape(C, 1)
    bg1c = bg1.reshape(2, 1)
    bg2c = bg2.reshape(2, 1)
    bfc = bfuse.reshape(C, 1)
    bl1c = bl1.reshape(C, 1)
    bl2c = bl2.reshape(C, 1)

    blk = lambda im: pl.BlockSpec((C, TW), im)
    tok, logit = pl.pallas_call(
        _ifm_body,
        grid=(G,),
        in_specs=[
            blk(lambda t: (0, jnp.maximum(t - 1, 0))),
            blk(lambda t: (0, t)),
            blk(lambda t: (0, jnp.minimum(t + 1, G - 1))),
            blk(lambda t: (0, jnp.maximum(t - 1, 0))),
            blk(lambda t: (0, t)),
            blk(lambda t: (0, jnp.minimum(t + 1, G - 1))),
            _full(Wq9), _full(bqc), _full(Wk9), _full(bkc),
            _full(Wg19), _full(bg1c), _full(Wg29), _full(bg2c),
            _full(Wfm), _full(bfc), _full(clip_col),
            _full(Wl1T), _full(bl1c), _full(Wl2T), _full(bl2c), _full(WgT),
        ],
        out_specs=[
            pl.BlockSpec((C, TW), lambda t: (0, t)),
            pl.BlockSpec((E, TW), lambda t: (0, t)),
        ],
        out_shape=[
            jax.ShapeDtypeStruct((C, N), jnp.bfloat16),
            jax.ShapeDtypeStruct((E, N), jnp.float32),
        ],
    )(xf, xf, xf, pf, pf, pf,
      Wq9, bqc, Wk9, bkc, Wg19, bg1c, Wg29, bg2c,
      Wfm, bfc, clip_col, Wl1T, bl1c, Wl2T, bl2c, WgT)

    y = pl.pallas_call(
        _moe_body,
        grid=(G,),
        in_specs=[
            pl.BlockSpec((C, TW), lambda t: (0, t)),
            pl.BlockSpec((E, TW), lambda t: (0, t)),
            _full(W1T), _full(b1c), _full(W2T), _full(b2c),
        ],
        out_specs=pl.BlockSpec((C, TW), lambda t: (0, t)),
        out_shape=jax.ShapeDtypeStruct((C, N), jnp.float32),
    )(tok, logit, W1T, b1c, W2T, b2c)

    return y.reshape(1, C, H, W)


# speedup vs baseline: 4.3935x; 1.4477x over previous
import jax
import jax.numpy as jnp
from jax import lax
from jax.experimental import pallas as pl

C = 96
E = 8
HID = 192
CLIP = 512
H = 224
W = 224
N = H * W
TW = 512
G = N // TW
AO = TW - 256
ATTW = TW + 512


def _ifm_body(xm, x0, xp, pm, p0, pp,
              Wq9, bq, Wk9, bk, Wg19, bg1, Wg29, bg2,
              Wfm, bfuse, clip_col, Wl1T, bl1, Wl2T, bl2, WgT,
              tok_ref, logit_ref):
    t = pl.program_id(0)
    base = t * TW
    win_x = jnp.concatenate([xm[...], x0[...], xp[...]], axis=1)
    win_p = jnp.concatenate([pm[...], p0[...], pp[...]], axis=1)

    def conv9(get_src, W9, b, o0, width, out_c):
        jj = lax.broadcasted_iota(jnp.int32, (1, width), 1)
        p = base + (o0 - TW) + jj
        colp = p % W
        acc = jnp.zeros((out_c, width), jnp.float32)
        for dx in (-1, 0, 1):
            a = jnp.zeros((out_c, width), jnp.float32)
            for dy in (-1, 0, 1):
                k = (dy + 1) * 3 + (dx + 1)
                s = dy * W + dx
                a = a + jnp.dot(W9[k], get_src(o0 + s, width),
                                preferred_element_type=jnp.float32)
            if dx == 0:
                acc = acc + a
            else:
                colq = colp + dx
                valid = (colq >= 0) & (colq < W)
                acc = acc + a * valid.astype(jnp.float32)
        return acc + b[...]

    x_src = lambda off, w: win_x[:, off:off + w]
    p_src = lambda off, w: win_p[:, off:off + w]

    xq = conv9(x_src, Wq9, bq, AO, ATTW, C)
    pk = conv9(p_src, Wk9, bk, AO, ATTW, C)
    att = jax.nn.sigmoid(xq * pk)
    ax = win_x[:, AO:AO + ATTW] * att
    ap = win_p[:, AO:AO + ATTW] * att

    xcat_src = lambda off, w: jnp.concatenate(
        [win_x[:, off:off + w], ax[:, off - AO:off - AO + w]], axis=0)
    pcat_src = lambda off, w: jnp.concatenate(
        [win_p[:, off:off + w], ap[:, off - AO:off - AO + w]], axis=0)

    xg = conv9(xcat_src, Wg19, bg1, TW, TW, 2)
    pg = conv9(pcat_src, Wg29, bg2, TW, TW, 2)

    xc = win_x[:, TW:2 * TW]
    axc = ax[:, TW - AO:2 * TW - AO]
    pc = win_p[:, TW:2 * TW]
    apc = ap[:, TW - AO:2 * TW - AO]
    x_out = xc * xg[0:1] + axc * xg[1:2]
    p_out = pc * pg[0:1] + apc * pg[1:2]

    fused = jnp.dot(Wfm[...], jnp.concatenate([x_out, p_out], axis=0),
                    preferred_element_type=jnp.float32) + bfuse[...]
    m1 = jnp.dot(Wl1T[...], clip_col[...],
                 preferred_element_type=jnp.float32) + bl1[...]
    m2 = jnp.dot(Wl2T[...], clip_col[...],
                 preferred_element_type=jnp.float32) + bl2[...]
    out2 = fused * m1 + m2 + fused
    tok_ref[...] = out2.astype(tok_ref.dtype)
    logit_ref[...] = jnp.dot(WgT[...], out2, preferred_element_type=jnp.float32)


def _moe_body(tok, logit, W1T, b1, W2T, b2, y_ref):
    lg = logit[...]
    eidx = lax.broadcasted_iota(jnp.int32, (E, TW), 0)
    m1 = jnp.max(lg, axis=0, keepdims=True)
    i1 = jnp.min(jnp.where(lg == m1, eidx, E), axis=0, keepdims=True)
    l2 = jnp.where(eidx == i1, -jnp.inf, lg)
    m2 = jnp.max(l2, axis=0, keepdims=True)
    i2 = jnp.min(jnp.where(l2 == m2, eidx, E), axis=0, keepdims=True)
    texp = jnp.exp(m2 - m1)
    w1 = 1.0 / (1.0 + texp)
    w2 = texp / (1.0 + texp)
    gates = jnp.where(eidx == i1, w1, 0.0) + jnp.where(eidx == i2, w2, 0.0)

    tt = tok[...]
    h = jnp.dot(W1T[...], tt, preferred_element_type=jnp.float32) + b1[...]
    h = jax.nn.gelu(h)
    hg = (h.reshape(E, HID, TW) * gates.reshape(E, 1, TW)).reshape(E * HID, TW)
    o = jnp.dot(W2T[...], hg.astype(jnp.bfloat16),
                preferred_element_type=jnp.float32)
    y_ref[...] = o + jnp.dot(b2[...], gates, preferred_element_type=jnp.float32)


def _full(arr):
    nd = arr.ndim
    return pl.BlockSpec(arr.shape, lambda t: (0,) * nd)


def kernel(x, prior, clip_feature, Wq, bq, Wk, bk, Wg1, bg1, Wg2, bg2,
           Wfuse, bfuse, Wl1, bl1, Wl2, bl2, Wgate, W1, b1, W2, b2):
    xf = jnp.pad(x.reshape(C, N), ((0, 0), (TW, TW)))
    pf = jnp.pad(prior.reshape(C, N), ((0, 0), (TW, TW)))
    Wq9 = jnp.transpose(Wq, (2, 3, 0, 1)).reshape(9, C, C)
    Wk9 = jnp.transpose(Wk, (2, 3, 0, 1)).reshape(9, C, C)
    Wg19 = jnp.transpose(Wg1, (2, 3, 0, 1)).reshape(9, 2, 2 * C)
    Wg29 = jnp.transpose(Wg2, (2, 3, 0, 1)).reshape(9, 2, 2 * C)
    Wfm = Wfuse.reshape(C, 2 * C)
    clip_col = clip_feature.reshape(CLIP, 1)
    Wl1T = jnp.transpose(Wl1)
    Wl2T = jnp.transpose(Wl2)
    WgT = jnp.transpose(Wgate)
    W1T = jnp.transpose(W1, (0, 2, 1)).reshape(E * HID, C).astype(jnp.bfloat16)
    W2T = jnp.transpose(W2, (2, 0, 1)).reshape(C, E * HID).astype(jnp.bfloat16)
    b1c = b1.reshape(E * HID, 1)
    b2c = jnp.transpose(b2)
    bqc = bq.reshape(C, 1)
    bkc = bk.reshape(C, 1)
    bg1c = bg1.reshape(2, 1)
    bg2c = bg2.reshape(2, 1)
    bfc = bfuse.reshape(C, 1)
    bl1c = bl1.reshape(C, 1)
    bl2c = bl2.reshape(C, 1)

    blk = lambda im: pl.BlockSpec((C, TW), im)
    tok, logit = pl.pallas_call(
        _ifm_body,
        grid=(G,),
        in_specs=[
            blk(lambda t: (0, t)),
            blk(lambda t: (0, t + 1)),
            blk(lambda t: (0, t + 2)),
            blk(lambda t: (0, t)),
            blk(lambda t: (0, t + 1)),
            blk(lambda t: (0, t + 2)),
            _full(Wq9), _full(bqc), _full(Wk9), _full(bkc),
            _full(Wg19), _full(bg1c), _full(Wg29), _full(bg2c),
            _full(Wfm), _full(bfc), _full(clip_col),
            _full(Wl1T), _full(bl1c), _full(Wl2T), _full(bl2c), _full(WgT),
        ],
        out_specs=[
            pl.BlockSpec((C, TW), lambda t: (0, t)),
            pl.BlockSpec((E, TW), lambda t: (0, t)),
        ],
        out_shape=[
            jax.ShapeDtypeStruct((C, N), jnp.bfloat16),
            jax.ShapeDtypeStruct((E, N), jnp.float32),
        ],
    )(xf, xf, xf, pf, pf, pf,
      Wq9, bqc, Wk9, bkc, Wg19, bg1c, Wg29, bg2c,
      Wfm, bfc, clip_col, Wl1T, bl1c, Wl2T, bl2c, WgT)

    y = pl.pallas_call(
        _moe_body,
        grid=(G,),
        in_specs=[
            pl.BlockSpec((C, TW), lambda t: (0, t)),
            pl.BlockSpec((E, TW), lambda t: (0, t)),
            _full(W1T), _full(b1c), _full(W2T), _full(b2c),
        ],
        out_specs=pl.BlockSpec((C, TW), lambda t: (0, t)),
        out_shape=jax.ShapeDtypeStruct((C, N), jnp.float32),
    )(tok, logit, W1T, b1c, W2T, b2c)

    return y.reshape(1, C, H, W)


# speedup vs baseline: 6.5767x; 1.4969x over previous
import functools

import jax
import jax.numpy as jnp
from jax import lax
from jax.experimental import pallas as pl
from jax.experimental.pallas import tpu as pltpu
from jax.experimental.pallas import tpu_sc as plsc

C = 96
E = 8
HID = 192
CLIP = 512
H = 224
W = 224
N = H * W
TW1 = 3584
G1 = N // TW1
TW = 1024
G = N // TW
HALO = TW1
AO = HALO - 256
ATTW = TW1 + 512


def _ifm_body(xm, x0, xp, pm, p0, pp,
              Wq9, bq, Wk9, bk, Wg19, bg1, Wg29, bg2,
              Wfm, bfuse, clip_col, Wl1T, bl1, Wl2T, bl2, WgT,
              tok_ref, logit_ref):
    t = pl.program_id(0)
    base = t * TW1
    jj3 = lax.broadcasted_iota(jnp.int32, (1, 3 * TW1), 1)
    q3 = base - TW1 + jj3
    wmask = ((q3 >= 0) & (q3 < N)).astype(jnp.float32)
    win_x = jnp.concatenate([xm[...], x0[...], xp[...]], axis=1) * wmask
    win_p = jnp.concatenate([pm[...], p0[...], pp[...]], axis=1) * wmask

    def conv9(get_src, W9, b, o0, width, out_c):
        jj = lax.broadcasted_iota(jnp.int32, (1, width), 1)
        p = base + (o0 - HALO) + jj
        colp = p % W
        acc = jnp.zeros((out_c, width), jnp.float32)
        for dx in (-1, 0, 1):
            a = jnp.zeros((out_c, width), jnp.float32)
            for dy in (-1, 0, 1):
                k = (dy + 1) * 3 + (dx + 1)
                s = dy * W + dx
                a = a + jnp.dot(W9[k], get_src(o0 + s, width),
                                preferred_element_type=jnp.float32)
            if dx == 0:
                acc = acc + a
            else:
                colq = colp + dx
                valid = (colq >= 0) & (colq < W)
                acc = acc + a * valid.astype(jnp.float32)
        return acc + b[...]

    x_src = lambda off, w: win_x[:, off:off + w]
    p_src = lambda off, w: win_p[:, off:off + w]

    xq = conv9(x_src, Wq9, bq, AO, ATTW, C)
    pk = conv9(p_src, Wk9, bk, AO, ATTW, C)
    att = jax.nn.sigmoid(xq * pk)
    ax = win_x[:, AO:AO + ATTW] * att
    ap = win_p[:, AO:AO + ATTW] * att

    xcat_src = lambda off, w: jnp.concatenate(
        [win_x[:, off:off + w], ax[:, off - AO:off - AO + w]], axis=0)
    pcat_src = lambda off, w: jnp.concatenate(
        [win_p[:, off:off + w], ap[:, off - AO:off - AO + w]], axis=0)

    xg = conv9(xcat_src, Wg19, bg1, HALO, TW1, 2)
    pg = conv9(pcat_src, Wg29, bg2, HALO, TW1, 2)

    xc = win_x[:, HALO:HALO + TW1]
    axc = ax[:, HALO - AO:HALO - AO + TW1]
    pc = win_p[:, HALO:HALO + TW1]
    apc = ap[:, HALO - AO:HALO - AO + TW1]
    x_out = xc * xg[0:1] + axc * xg[1:2]
    p_out = pc * pg[0:1] + apc * pg[1:2]

    fused = jnp.dot(Wfm[...], jnp.concatenate([x_out, p_out], axis=0),
                    preferred_element_type=jnp.float32) + bfuse[...]
    m1 = jnp.dot(Wl1T[...], clip_col[...],
                 preferred_element_type=jnp.float32) + bl1[...]
    m2 = jnp.dot(Wl2T[...], clip_col[...],
                 preferred_element_type=jnp.float32) + bl2[...]
    out2 = fused * m1 + m2 + fused
    tok_ref[...] = out2.astype(tok_ref.dtype)
    logit_ref[...] = jnp.dot(WgT[...], out2, preferred_element_type=jnp.float32)


NC = 2
NS = 16
NW = NC * NS
LPW = N // NW
SCL = 16


def _route_body(lg_hbm, g_hbm, lbuf, gbuf):
    wid = lax.axis_index("s") * NC + lax.axis_index("c")
    for e in range(E):
        pltpu.sync_copy(lg_hbm.at[e, wid], lbuf.at[e])

    def step(i, _):
        sl = pl.ds(i * SCL, SCL)
        ls = [lbuf[e, sl] for e in range(E)]
        m1 = ls[0]
        for e in range(1, E):
            m1 = jnp.maximum(m1, ls[e])
        i1 = jnp.full((SCL,), E, jnp.int32)
        for e in range(E - 1, -1, -1):
            i1 = jnp.where(ls[e] == m1, e, i1)
        l2 = [jnp.where(i1 == e, -jnp.inf, ls[e]) for e in range(E)]
        m2 = l2[0]
        for e in range(1, E):
            m2 = jnp.maximum(m2, l2[e])
        i2 = jnp.full((SCL,), E, jnp.int32)
        for e in range(E - 1, -1, -1):
            i2 = jnp.where(l2[e] == m2, e, i2)
        t = jnp.exp(m2 - m1)
        den = 1.0 + t
        w1 = 1.0 / den
        w2 = t / den
        for e in range(E):
            gbuf[e, sl] = (jnp.where(i1 == e, w1, 0.0)
                           + jnp.where(i2 == e, w2, 0.0))
        return 0

    lax.fori_loop(0, LPW // SCL, step, 0)
    for e in range(E):
        pltpu.sync_copy(gbuf.at[e], g_hbm.at[e, wid])


def _route(logit):
    lg = logit.reshape(E, NW, LPW)
    mesh = plsc.VectorSubcoreMesh(core_axis_name="c", subcore_axis_name="s")
    f = functools.partial(
        pl.kernel, mesh=mesh,
        out_type=jax.ShapeDtypeStruct((E, NW, LPW), jnp.float32),
        scratch_types=[
            pltpu.VMEM((E, LPW), jnp.float32),
            pltpu.VMEM((E, LPW), jnp.float32),
        ],
    )(_route_body)
    return f(lg).reshape(E, N)


def _moe_body(tok, gate, W1T, b1, W2T, b2, y_ref):
    gates = gate[...]
    tt = tok[...]
    h = jnp.dot(W1T[...], tt, preferred_element_type=jnp.float32) + b1[...]
    h = jax.nn.gelu(h.astype(jnp.bfloat16))
    gb = gates.astype(jnp.bfloat16)
    hg = (h.reshape(E, HID, TW) * gb.reshape(E, 1, TW)).reshape(E * HID, TW)
    o = jnp.dot(W2T[...], hg, preferred_element_type=jnp.float32)
    y_ref[...] = o + jnp.dot(b2[...], gates, preferred_element_type=jnp.float32)


def _full(arr):
    nd = arr.ndim
    return pl.BlockSpec(arr.shape, lambda t: (0,) * nd)


def kernel(x, prior, clip_feature, Wq, bq, Wk, bk, Wg1, bg1, Wg2, bg2,
           Wfuse, bfuse, Wl1, bl1, Wl2, bl2, Wgate, W1, b1, W2, b2):
    xf = x.reshape(C, N)
    pf = prior.reshape(C, N)
    Wq9 = jnp.transpose(Wq, (2, 3, 0, 1)).reshape(9, C, C)
    Wk9 = jnp.transpose(Wk, (2, 3, 0, 1)).reshape(9, C, C)
    Wg19 = jnp.transpose(Wg1, (2, 3, 0, 1)).reshape(9, 2, 2 * C)
    Wg29 = jnp.transpose(Wg2, (2, 3, 0, 1)).reshape(9, 2, 2 * C)
    Wfm = Wfuse.reshape(C, 2 * C)
    clip_col = clip_feature.reshape(CLIP, 1)
    Wl1T = jnp.transpose(Wl1)
    Wl2T = jnp.transpose(Wl2)
    WgT = jnp.transpose(Wgate)
    W1T = jnp.transpose(W1, (0, 2, 1)).reshape(E * HID, C).astype(jnp.bfloat16)
    W2T = jnp.transpose(W2, (2, 0, 1)).reshape(C, E * HID).astype(jnp.bfloat16)
    b1c = b1.reshape(E * HID, 1)
    b2c = jnp.transpose(b2)
    bqc = bq.reshape(C, 1)
    bkc = bk.reshape(C, 1)
    bg1c = bg1.reshape(2, 1)
    bg2c = bg2.reshape(2, 1)
    bfc = bfuse.reshape(C, 1)
    bl1c = bl1.reshape(C, 1)
    bl2c = bl2.reshape(C, 1)

    blk = lambda im: pl.BlockSpec((C, TW1), im)
    tok, logit = pl.pallas_call(
        _ifm_body,
        grid=(G1,),
        in_specs=[
            blk(lambda t: (0, jnp.maximum(t - 1, 0))),
            blk(lambda t: (0, t)),
            blk(lambda t: (0, jnp.minimum(t + 1, G1 - 1))),
            blk(lambda t: (0, jnp.maximum(t - 1, 0))),
            blk(lambda t: (0, t)),
            blk(lambda t: (0, jnp.minimum(t + 1, G1 - 1))),
            _full(Wq9), _full(bqc), _full(Wk9), _full(bkc),
            _full(Wg19), _full(bg1c), _full(Wg29), _full(bg2c),
            _full(Wfm), _full(bfc), _full(clip_col),
            _full(Wl1T), _full(bl1c), _full(Wl2T), _full(bl2c), _full(WgT),
        ],
        out_specs=[
            pl.BlockSpec((C, TW1), lambda t: (0, t)),
            pl.BlockSpec((E, TW1), lambda t: (0, t)),
        ],
        out_shape=[
            jax.ShapeDtypeStruct((C, N), jnp.bfloat16),
            jax.ShapeDtypeStruct((E, N), jnp.float32),
        ],
    )(xf, xf, xf, pf, pf, pf,
      Wq9, bqc, Wk9, bkc, Wg19, bg1c, Wg29, bg2c,
      Wfm, bfc, clip_col, Wl1T, bl1c, Wl2T, bl2c, WgT)

    gates = _route(logit)

    y = pl.pallas_call(
        _moe_body,
        grid=(G,),
        in_specs=[
            pl.BlockSpec((C, TW), lambda t: (0, t)),
            pl.BlockSpec((E, TW), lambda t: (0, t)),
            _full(W1T), _full(b1c), _full(W2T), _full(b2c),
        ],
        out_specs=pl.BlockSpec((C, TW), lambda t: (0, t)),
        out_shape=jax.ShapeDtypeStruct((C, N), jnp.float32),
    )(tok, gates, W1T, b1c, W2T, b2c)

    return y.reshape(1, C, H, W)
